# baseline (speedup 1.0000x reference)
import jax
import jax.numpy as jnp
from jax.experimental import pallas as pl

HEIGHT = 64
WIDTH = 128
EPS = 1e-6
B_BLK = 512
LANES = 128


def _som_kernel(x_ref, w2x_ref, x2_ref, sx_ref, w2_ref, sw_ref, loc_ref, loss_ref):
    x = x_ref[:]
    Bb, V = x.shape
    K = w2x_ref.shape[0]
    G = K // LANES
    x2 = x2_ref[:]
    sx = sx_ref[:]
    w2 = w2_ref[:]
    sw = sw_ref[:]
    t1 = jax.lax.dot_general(x, w2x_ref[:], (((1,), (1,)), ((), ())),
                             preferred_element_type=jnp.float32)

    m_run = jnp.full((Bb, LANES), jnp.inf, dtype=jnp.float32)
    g_run = jnp.zeros((Bb, LANES), dtype=jnp.float32)
    for g in range(G):
        lo = g * LANES
        d2 = (x2 + w2[:, lo:lo + LANES] - t1[:, lo:lo + LANES]
              + 2.0 * EPS * (sx - sw[:, lo:lo + LANES]) + V * EPS * EPS)
        d2 = jnp.maximum(d2, 0.0)
        c = d2 < m_run
        m_run = jnp.minimum(m_run, d2)
        g_run = jnp.where(c, jnp.float32(g), g_run)

    m = jnp.min(m_run, axis=1, keepdims=True)
    loss_ref[:] = jnp.sqrt(m[:, 0])
    lane = (jax.lax.broadcasted_iota(jnp.int32, (Bb, LANES), 1)
            .astype(jnp.float32))
    kcand = jnp.where(m_run == m, g_run * float(LANES) + lane, float(K))
    idx = jnp.min(kcand, axis=1)
    ii = jnp.floor(idx * (1.0 / WIDTH))
    jj = idx - ii * WIDTH
    loc_ref[:, 0] = ii
    loc_ref[:, 1] = jj


def kernel(input, weights, locations):
    B, V = input.shape
    K = weights.shape[0]
    n_blk = B // B_BLK
    x2 = jnp.sum(input * input, axis=1, keepdims=True)
    sx = jnp.sum(input, axis=1, keepdims=True)
    w2 = jnp.sum(weights * weights, axis=1)[None, :]
    sw = jnp.sum(weights, axis=1)[None, :]
    w2x = weights + weights
    loc, losses = pl.pallas_call(
        _som_kernel,
        grid=(n_blk,),
        in_specs=[
            pl.BlockSpec((B_BLK, V), lambda i: (i, 0)),
            pl.BlockSpec((K, V), lambda i: (0, 0)),
            pl.BlockSpec((B_BLK, 1), lambda i: (i, 0)),
            pl.BlockSpec((B_BLK, 1), lambda i: (i, 0)),
            pl.BlockSpec((1, K), lambda i: (0, 0)),
            pl.BlockSpec((1, K), lambda i: (0, 0)),
        ],
        out_specs=[
            pl.BlockSpec((B_BLK, 2), lambda i: (i, 0)),
            pl.BlockSpec((B_BLK,), lambda i: (i,)),
        ],
        out_shape=[
            jax.ShapeDtypeStruct((B, 2), jnp.float32),
            jax.ShapeDtypeStruct((B,), jnp.float32),
        ],
    )(input, w2x, x2, sx, w2, sw)
    return (loc, losses)


# speedup vs baseline: 1.1059x; 1.1059x over previous
import jax
import jax.numpy as jnp
from jax.experimental import pallas as pl

HEIGHT = 64
WIDTH = 128
EPS = 1e-6
B_BLK = 1024
K_CHUNKS = 1


def _som_kernel(x_ref, w2x_ref, x2_ref, sx_ref, w2_ref, sw_ref, loc_ref, loss_ref):
    x = x_ref[:]
    V = x.shape[1]
    Bb = x.shape[0]
    K = w2x_ref.shape[0]
    KC = K // K_CHUNKS
    x2 = x2_ref[:]
    sx = sx_ref[:]

    chunk_min = []
    chunk_idx = []
    for c in range(K_CHUNKS):
        w2x = w2x_ref[pl.ds(c * KC, KC), :]
        w2 = w2_ref[:, pl.ds(c * KC, KC)]
        sw = sw_ref[:, pl.ds(c * KC, KC)]
        t1 = jax.lax.dot_general(x, w2x, (((1,), (1,)), ((), ())),
                                 preferred_element_type=jnp.float32)
        d2 = x2 + w2 - t1 + 2.0 * EPS * (sx - sw) + V * EPS * EPS
        m = jnp.maximum(jnp.min(d2, axis=1, keepdims=True), 0.0)
        kidx = (jax.lax.broadcasted_iota(jnp.int32, d2.shape, 1)
                .astype(jnp.float32)) + float(c * KC)
        idx = jnp.min(jnp.where(d2 <= m, kidx, float(K)), axis=1,
                      keepdims=True)
        chunk_min.append(m)
        chunk_idx.append(idx)

    ms = jnp.concatenate(chunk_min, axis=1)
    idxs = jnp.concatenate(chunk_idx, axis=1)
    m = jnp.min(ms, axis=1, keepdims=True)
    loss_ref[:] = jnp.sqrt(m[:, 0])
    idx = jnp.min(jnp.where(ms == m, idxs, float(K)), axis=1)
    ii = jnp.floor(idx * (1.0 / WIDTH))
    jj = idx - ii * WIDTH
    loc_ref[:, 0] = ii
    loc_ref[:, 1] = jj


def kernel(input, weights, locations):
    B, V = input.shape
    K = weights.shape[0]
    n_blk = B // B_BLK
    x2 = jnp.sum(input * input, axis=1, keepdims=True)
    sx = jnp.sum(input, axis=1, keepdims=True)
    w2 = jnp.sum(weights * weights, axis=1)[None, :]
    sw = jnp.sum(weights, axis=1)[None, :]
    w2x = weights + weights
    loc, losses = pl.pallas_call(
        _som_kernel,
        grid=(n_blk,),
        in_specs=[
            pl.BlockSpec((B_BLK, V), lambda i: (i, 0)),
            pl.BlockSpec((K, V), lambda i: (0, 0)),
            pl.BlockSpec((B_BLK, 1), lambda i: (i, 0)),
            pl.BlockSpec((B_BLK, 1), lambda i: (i, 0)),
            pl.BlockSpec((1, K), lambda i: (0, 0)),
            pl.BlockSpec((1, K), lambda i: (0, 0)),
        ],
        out_specs=[
            pl.BlockSpec((B_BLK, 2), lambda i: (i, 0)),
            pl.BlockSpec((B_BLK,), lambda i: (i,)),
        ],
        out_shape=[
            jax.ShapeDtypeStruct((B, 2), jnp.float32),
            jax.ShapeDtypeStruct((B,), jnp.float32),
        ],
    )(input, w2x, x2, sx, w2, sw)
    return (loc, losses)
